# baseline (speedup 1.0000x reference)
import functools

import jax
import jax.numpy as jnp
from jax import lax
from jax.experimental import pallas as pl
from jax.experimental.pallas import tpu as pltpu
from jax.experimental.pallas import tpu_sc as plsc

_B = 4
_S = 2048
_D = 768
_N = _B * _S
_L = 16
_NC = 2
_NS = 16
_NW = _NC * _NS
_PER_W = _N // _NW
_C = 32
_NCH = _PER_W // _C
_CVECS = _D // _L


def _make_sc_embed():
    mesh = plsc.VectorSubcoreMesh(core_axis_name="c", subcore_axis_name="s")

    @functools.partial(
        pl.kernel,
        mesh=mesh,
        out_type=jax.ShapeDtypeStruct((_N, _D), jnp.float32),
        scratch_types=[
            pltpu.VMEM((_PER_W,), jnp.int32),
            pltpu.VMEM((_PER_W,), jnp.int32),
            pltpu.VMEM((_C, _D), jnp.float32),
            pltpu.VMEM((_C, _D), jnp.float32),
            pltpu.VMEM((_C, _D), jnp.float32),
            pltpu.SemaphoreType.DMA,
            pltpu.SemaphoreType.DMA,
            pltpu.SemaphoreType.DMA,
        ],
    )
    def sc_embed(ids_hbm, sids_hbm, tok_hbm, seg_hbm, pos_hbm, out_hbm,
                 idx_v, sidx_v, tok_v, seg_v, pos_v, sem_t, sem_s, sem_p):
        wid = lax.axis_index("s") * _NC + lax.axis_index("c")
        base = wid * _PER_W
        s0 = base % _S

        pltpu.sync_copy(ids_hbm.at[pl.ds(base, _PER_W)], idx_v)
        pltpu.sync_copy(sids_hbm.at[pl.ds(base, _PER_W)], sidx_v)

        def chunk_body(j, carry):
            cp_t = pltpu.async_copy(
                tok_hbm.at[idx_v.at[pl.ds(j * _C, _C)]], tok_v, sem_t)
            cp_s = pltpu.async_copy(
                seg_hbm.at[sidx_v.at[pl.ds(j * _C, _C)]], seg_v, sem_s)
            cp_p = pltpu.async_copy(
                pos_hbm.at[pl.ds(s0 + j * _C, _C)], pos_v, sem_p)
            cp_t.wait()
            cp_s.wait()
            cp_p.wait()

            def row_body(r, carry2):
                def vec_body(cb, carry3):
                    sl = pl.ds(cb * _L, _L)
                    tok_v[r, sl] = tok_v[r, sl] + seg_v[r, sl] + pos_v[r, sl]
                    return carry3
                return lax.fori_loop(0, _CVECS, vec_body, carry2)

            lax.fori_loop(0, _C, row_body, None)
            pltpu.sync_copy(tok_v, out_hbm.at[pl.ds(base + j * _C, _C)])
            return carry

        lax.fori_loop(0, _NCH, chunk_body, None)

    return sc_embed


_sc_embed = _make_sc_embed()


@jax.jit
def kernel(input_ids, segment_ids, token_table, segment_table,
           position_table):
    ids = input_ids.reshape(-1).astype(jnp.int32)
    sids = segment_ids.reshape(-1).astype(jnp.int32)
    out = _sc_embed(ids, sids, token_table, segment_table, position_table)
    return out.reshape(_B, _S, _D)


# speedup vs baseline: 1.0831x; 1.0831x over previous
import functools

import jax
import jax.numpy as jnp
from jax import lax
from jax.experimental import pallas as pl
from jax.experimental.pallas import tpu as pltpu
from jax.experimental.pallas import tpu_sc as plsc

_B = 4
_S = 2048
_D = 768
_N = _B * _S
_L = 16
_NC = 2
_NS = 16
_NW = _NC * _NS
_PER_W = _N // _NW
_C = 32
_NCH = _PER_W // _C
_CVECS = _D // _L


def _make_sc_embed():
    mesh = plsc.VectorSubcoreMesh(core_axis_name="c", subcore_axis_name="s")

    @functools.partial(
        pl.kernel,
        mesh=mesh,
        out_type=jax.ShapeDtypeStruct((_N, _D), jnp.float32),
        scratch_types=[
            pltpu.VMEM((_PER_W,), jnp.int32),
            pltpu.VMEM((_PER_W,), jnp.int32),
            pltpu.VMEM((_C, _D), jnp.float32),
            pltpu.VMEM((_C, _D), jnp.float32),
            pltpu.VMEM((_C, _D), jnp.float32),
            pltpu.SemaphoreType.DMA,
            pltpu.SemaphoreType.DMA,
            pltpu.SemaphoreType.DMA,
        ],
    )
    def sc_embed(ids_hbm, sids_hbm, tok_hbm, seg_hbm, pos_hbm, out_hbm,
                 idx_v, sidx_v, tok_v, seg_v, pos_v, sem_t, sem_s, sem_p):
        wid = lax.axis_index("s") * _NC + lax.axis_index("c")
        base = wid * _PER_W
        s0 = base % _S

        pltpu.sync_copy(ids_hbm.at[pl.ds(base, _PER_W)], idx_v)
        pltpu.sync_copy(sids_hbm.at[pl.ds(base, _PER_W)], sidx_v)

        def chunk_body(j, carry):
            cp_t = pltpu.async_copy(
                tok_hbm.at[idx_v.at[pl.ds(j * _C, _C)]], tok_v, sem_t)
            cp_s = pltpu.async_copy(
                seg_hbm.at[sidx_v.at[pl.ds(j * _C, _C)]], seg_v, sem_s)
            cp_p = pltpu.async_copy(
                pos_hbm.at[pl.ds(s0 + j * _C, _C)], pos_v, sem_p)
            cp_t.wait()
            cp_s.wait()
            cp_p.wait()

            def row_body(r, carry2):
                for cb in range(_CVECS):
                    sl = pl.ds(cb * _L, _L)
                    tok_v[r, sl] = tok_v[r, sl] + seg_v[r, sl] + pos_v[r, sl]
                return carry2

            lax.fori_loop(0, _C, row_body, None)
            pltpu.sync_copy(tok_v, out_hbm.at[pl.ds(base + j * _C, _C)])
            return carry

        lax.fori_loop(0, _NCH, chunk_body, None)

    return sc_embed


_sc_embed = _make_sc_embed()


@jax.jit
def kernel(input_ids, segment_ids, token_table, segment_table,
           position_table):
    ids = input_ids.reshape(-1).astype(jnp.int32)
    sids = segment_ids.reshape(-1).astype(jnp.int32)
    out = _sc_embed(ids, sids, token_table, segment_table, position_table)
    return out.reshape(_B, _S, _D)
